# baseline (speedup 1.0000x reference)
import functools

import jax
import jax.numpy as jnp
from jax import lax
from jax.experimental import pallas as pl
from jax.experimental.pallas import tpu as pltpu, tpu_sc as plsc

NC = 2
NS = 16
NW = NC * NS
LANES = 16

N_NODES = 10000
N_PAD = 10240
SEG = N_PAD // NS
E_TOT = 320000
E_W = E_TOT // NW
CW = 80
NCHUNK = E_W // CW
ROWS_W = N_PAD // NS
D = 128
G = 64
ROWBLK = 128

_mesh = plsc.VectorSubcoreMesh(core_axis_name="c", subcore_axis_name="s")


def _zero_vmem_1d(ref, n):
    z16 = jnp.zeros((LANES,), jnp.float32)

    def body(i, carry):
        ref[pl.ds(i * LANES, LANES)] = z16
        return carry

    lax.fori_loop(0, n // LANES, body, 0)


def _rsqrt16(x):
    i = plsc.bitcast(x, jnp.int32)
    y = plsc.bitcast(jnp.int32(0x5F3759DF) - (i >> 1), jnp.float32)
    for _ in range(4):
        y = y * (1.5 - 0.5 * x * y * y)
    return y


@functools.partial(
    pl.kernel,
    out_type=[
        jax.ShapeDtypeStruct((NW, NCHUNK, CW), jnp.float32),
        jax.ShapeDtypeStruct((NW, NCHUNK, CW), jnp.int32),
        jax.ShapeDtypeStruct((N_PAD,), jnp.float32),
    ],
    mesh=_mesh,
    compiler_params=pltpu.CompilerParams(needs_layout_passes=False),
    scratch_types=[
        pltpu.VMEM((NCHUNK, CW), jnp.int32),
        pltpu.VMEM((NCHUNK, CW), jnp.float32),
        pltpu.VMEM((N_PAD,), jnp.float32),
        pltpu.VMEM((SEG,), jnp.float32),
        pltpu.VMEM((SEG,), jnp.float32),
        pltpu.VMEM((N_PAD,), jnp.float32),
        pltpu.VMEM((N_NODES,), jnp.int32),
        pltpu.VMEM((NCHUNK, CW), jnp.int32),
        pltpu.VMEM((NCHUNK, CW), jnp.float32),
        pltpu.VMEM((NCHUNK, CW), jnp.int32),
        pltpu.VMEM_SHARED((NS, N_PAD), jnp.float32),
        pltpu.VMEM_SHARED((N_PAD,), jnp.float32),
    ],
)
def _k0(src3, dst3, ew3, batch_t, norm_o, bidx_o, selfn_o,
        dstA, ewA, degv, colacc, segbuf, dinvv, batchv,
        srcC, normv, bidxv, stage, dinv_sh):
    c = lax.axis_index("c")
    s = lax.axis_index("s")
    wid = c * NS + s

    _zero_vmem_1d(degv, N_PAD)

    def acc_deg(chunk_id):
        pltpu.sync_copy(dst3.at[chunk_id], dstA)
        pltpu.sync_copy(ew3.at[chunk_id], ewA)

        def body(j, carry):
            for k in range(CW // LANES):
                d16 = dstA[j, pl.ds(k * LANES, LANES)]
                w16 = ewA[j, pl.ds(k * LANES, LANES)]
                plsc.addupdate_scatter(degv, [d16], w16)
            return carry

        lax.fori_loop(0, NCHUNK, body, 0)

    acc_deg(s)
    acc_deg(s + NS)

    pltpu.sync_copy(degv, stage.at[s])
    plsc.subcore_barrier()
    _zero_vmem_1d(colacc, SEG)

    def reduce_tile(t, carry):
        pltpu.sync_copy(stage.at[t, pl.ds(s * SEG, SEG)], segbuf)
        for m in range(SEG // LANES):
            sl = pl.ds(m * LANES, LANES)
            colacc[sl] = colacc[sl] + segbuf[sl]
        return carry

    lax.fori_loop(0, NS, reduce_tile, 0)

    for m in range(SEG // LANES):
        sl = pl.ds(m * LANES, LANES)
        x = colacc[sl] + 1.0
        y = _rsqrt16(x)
        segbuf[sl] = y
        colacc[sl] = y * y

    pltpu.sync_copy(segbuf, dinv_sh.at[pl.ds(s * SEG, SEG)])

    @pl.when(c == 0)
    def _():
        pltpu.sync_copy(colacc, selfn_o.at[pl.ds(s * SEG, SEG)])

    plsc.subcore_barrier()
    pltpu.sync_copy(dinv_sh, dinvv)

    pltpu.sync_copy(batch_t, batchv)
    pltpu.sync_copy(src3.at[wid], srcC)
    pltpu.sync_copy(dst3.at[wid], dstA)
    pltpu.sync_copy(ew3.at[wid], ewA)

    def norm_body(j, carry):
        for k in range(CW // LANES):
            sl = pl.ds(k * LANES, LANES)
            s16 = srcC[j, sl]
            d16 = dstA[j, sl]
            w16 = ewA[j, sl]
            gs = plsc.load_gather(dinvv, [s16])
            gd = plsc.load_gather(dinvv, [d16])
            normv[j, sl] = gs * w16 * gd
            bidxv[j, sl] = plsc.load_gather(batchv, [d16])
        return carry

    lax.fori_loop(0, NCHUNK, norm_body, 0)
    pltpu.sync_copy(normv, norm_o.at[wid])
    pltpu.sync_copy(bidxv, bidx_o.at[wid])


CWA = 112
NCH_A = 90
E_WP = NCH_A * CWA
WB = 80


@functools.partial(
    pl.kernel,
    out_type=jax.ShapeDtypeStruct((NC, N_PAD, D), jnp.float32),
    mesh=_mesh,
    compiler_params=pltpu.CompilerParams(needs_layout_passes=False),
    scratch_types=(
        [pltpu.VMEM((CWA,), jnp.int32)] * 3
        + [pltpu.VMEM((CWA,), jnp.float32)] * 3
        + [pltpu.VMEM((CWA,), jnp.int32)] * 3
        + [pltpu.VMEM((CWA, D), jnp.float32)] * 3
        + [pltpu.SemaphoreType.DMA] * 6
        + [pltpu.VMEM_SHARED((N_PAD, D), jnp.float32)]
    ),
)
def _agg(hw, srcp, dstp, normp, out,
         sw0, sw1, sw2, nw0, nw1, nw2, dw0, dw1, dw2, rw0, rw1, rw2,
         gs0, gs1, gs2, ss0, ss1, ss2, acc):
    c = lax.axis_index("c")
    s = lax.axis_index("s")
    wid = c * NS + s
    ebase = wid * E_WP
    bufs = [(sw0, nw0, dw0, rw0, gs0, ss0),
            (sw1, nw1, dw1, rw1, gs1, ss1),
            (sw2, nw2, dw2, rw2, gs2, ss2)]

    z16 = jnp.zeros((LANES,), jnp.float32)

    def zblk(j, carry):
        for k in range(D // LANES):
            rw0[j, pl.ds(k * LANES, LANES)] = z16
        return carry

    lax.fori_loop(0, CWA, zblk, 0)
    for i in range(ROWS_W // WB):
        pltpu.sync_copy(rw0.at[pl.ds(0, WB)],
                        acc.at[pl.ds(s * ROWS_W + i * WB, WB)])
    plsc.subcore_barrier()

    def win_copy(j, sw, nw, dw):
        e0 = ebase + j * CWA
        pltpu.sync_copy(srcp.at[pl.ds(e0, CWA)], sw)
        pltpu.sync_copy(normp.at[pl.ds(e0, CWA)], nw)
        pltpu.sync_copy(dstp.at[pl.ds(e0, CWA)], dw)

    def scale(rows, nw):
        @plsc.parallel_loop(0, CWA // LANES, unroll=2)
        def grp(g):
            nm16 = nw[pl.ds(g * LANES, LANES)]
            for l in range(LANES):
                e = g * LANES + l
                sc = jnp.broadcast_to(nm16[l], (LANES,))
                for k in range(D // LANES):
                    sl = pl.ds(k * LANES, LANES)
                    rows[e, sl] = rows[e, sl] * sc

    win_copy(0, sw0, nw0, dw0)
    pltpu.async_copy(hw.at[sw0], rw0, gs0)
    win_copy(1, sw1, nw1, dw1)
    pltpu.async_copy(hw.at[sw1], rw1, gs1)

    def body(i, carry):
        for t in range(3):
            sw, nw, dw, rw, gs, ss = bufs[t]
            j = 3 * i + t
            nt = (t + 2) % 3
            swn, nwn, dwn, rwn, gsn, ssn = bufs[nt]
            if t == 0:
                @pl.when(i > 0)
                def _():
                    pltpu.make_async_copy(rwn, acc.at[dwn], ssn).wait()

                win_copy(j + 2, swn, nwn, dwn)
                pltpu.async_copy(hw.at[swn], rwn, gsn)
            else:
                @pl.when(j + 2 < NCH_A)
                def _():
                    pltpu.make_async_copy(rwn, acc.at[dwn], ssn).wait()
                    win_copy(j + 2, swn, nwn, dwn)
                    pltpu.async_copy(hw.at[swn], rwn, gsn)

            pltpu.make_async_copy(hw.at[sw], rw, gs).wait()
            scale(rw, nw)
            pltpu.async_copy(rw, acc.at[dw], ss, add=True)
        return carry

    lax.fori_loop(0, NCH_A // 3, body, 0)
    for t in range(3):
        sw, nw, dw, rw, gs, ss = bufs[t]
        pltpu.make_async_copy(rw, acc.at[dw], ss).wait()
    plsc.subcore_barrier()

    for i in range(ROWS_W // WB):
        r0 = s * ROWS_W + i * WB
        pltpu.sync_copy(acc.at[pl.ds(r0, WB)], rw0.at[pl.ds(0, WB)])
        pltpu.sync_copy(rw0.at[pl.ds(0, WB)], out.at[c, pl.ds(r0, WB)])


@functools.partial(
    pl.kernel,
    out_type=jax.ShapeDtypeStruct((NW, G), jnp.float32),
    mesh=_mesh,
    compiler_params=pltpu.CompilerParams(needs_layout_passes=False),
    scratch_types=[
        pltpu.VMEM((N_PAD,), jnp.float32),
        pltpu.VMEM((E_W,), jnp.int32),
        pltpu.VMEM((E_W,), jnp.float32),
        pltpu.VMEM((E_W,), jnp.int32),
        pltpu.VMEM((G,), jnp.float32),
    ],
)
def _pool(vtab, src1, norm1, bidx1, out, vv, srcv, normv, bidxv, bins):
    c = lax.axis_index("c")
    s = lax.axis_index("s")
    wid = c * NS + s
    ebase = wid * E_W
    pltpu.sync_copy(vtab, vv)
    pltpu.sync_copy(src1.at[pl.ds(ebase, E_W)], srcv)
    pltpu.sync_copy(norm1.at[pl.ds(ebase, E_W)], normv)
    pltpu.sync_copy(bidx1.at[pl.ds(ebase, E_W)], bidxv)
    _zero_vmem_1d(bins, G)

    def body(j, carry):
        sl = pl.ds(j * LANES, LANES)
        s16 = srcv[sl]
        nm = normv[sl]
        b16 = bidxv[sl]
        gv = plsc.load_gather(vv, [s16])
        plsc.addupdate_scatter(bins, [b16], gv * nm)
        return carry

    lax.fori_loop(0, E_W // LANES, body, 0)
    pltpu.sync_copy(bins, out.at[wid])


_RB = 2048
_NG = N_PAD // _RB


def _mm_body(x_ref, w_ref, o_ref):
    o_ref[...] = jnp.dot(x_ref[...], w_ref[...],
                         preferred_element_type=jnp.float32)


def _mm(x, w):
    return pl.pallas_call(
        _mm_body,
        grid=(_NG,),
        in_specs=[
            pl.BlockSpec((_RB, D), lambda i: (i, 0)),
            pl.BlockSpec((D, D), lambda i: (0, 0)),
        ],
        out_specs=pl.BlockSpec((_RB, D), lambda i: (i, 0)),
        out_shape=jax.ShapeDtypeStruct((N_PAD, D), jnp.float32),
    )(x, w)


def _comb_body(p_ref, hw_ref, sn_ref, b_ref, w_ref, o_ref):
    h = p_ref[0] + p_ref[1] + hw_ref[...] * sn_ref[...] + b_ref[...]
    h = jnp.maximum(h, 0.0)
    o_ref[...] = jnp.dot(h, w_ref[...], preferred_element_type=jnp.float32)


def _comb_mm(p, hw, sn, b, w):
    return pl.pallas_call(
        _comb_body,
        grid=(_NG,),
        in_specs=[
            pl.BlockSpec((NC, _RB, D), lambda i: (0, i, 0)),
            pl.BlockSpec((_RB, D), lambda i: (i, 0)),
            pl.BlockSpec((_RB, 1), lambda i: (i, 0)),
            pl.BlockSpec((1, D), lambda i: (0, 0)),
            pl.BlockSpec((D, D), lambda i: (0, 0)),
        ],
        out_specs=pl.BlockSpec((_RB, D), lambda i: (i, 0)),
        out_shape=jax.ShapeDtypeStruct((N_PAD, D), jnp.float32),
    )(p, hw, sn, b, w)


def _head_body(q_ref, hw_ref, sn_ref, b_ref, w3_ref, wl_ref, o_ref):
    h = q_ref[0] + q_ref[1] + hw_ref[...] * sn_ref[...] + b_ref[...]
    h = jnp.maximum(h, 0.0)
    w3l = jnp.dot(w3_ref[...], wl_ref[...], preferred_element_type=jnp.float32)
    o_ref[...] = jnp.dot(h, w3l, preferred_element_type=jnp.float32)


def _head(q, hw, sn, b, w3, wl):
    return pl.pallas_call(
        _head_body,
        grid=(_NG,),
        in_specs=[
            pl.BlockSpec((NC, _RB, D), lambda i: (0, i, 0)),
            pl.BlockSpec((_RB, D), lambda i: (i, 0)),
            pl.BlockSpec((_RB, 1), lambda i: (i, 0)),
            pl.BlockSpec((1, D), lambda i: (0, 0)),
            pl.BlockSpec((D, D), lambda i: (0, 0)),
            pl.BlockSpec((D, 1), lambda i: (0, 0)),
        ],
        out_specs=pl.BlockSpec((_RB, 1), lambda i: (i, 0)),
        out_shape=jax.ShapeDtypeStruct((N_PAD, 1), jnp.float32),
    )(q, hw, sn, b, w3, wl)


_NROW2D = N_PAD // D


def _final_body(batch_ref, v_ref, sn_ref, zp_ref, b3_ref, wl_ref, bl_ref,
                o_ref):
    giota = lax.broadcasted_iota(jnp.int32, (G, 1), 0)
    liota = lax.broadcasted_iota(jnp.int32, (1, D), 1)

    def row(r, carry):
        sums, cnts = carry
        brow = batch_ref[pl.ds(r, 1), :]
        wrow = v_ref[pl.ds(r, 1), :] * sn_ref[pl.ds(r, 1), :]
        valid = ((liota + r * D) < N_NODES).astype(jnp.float32)
        m = (brow == giota).astype(jnp.float32)
        sums = sums + jnp.sum(m * wrow, axis=1, keepdims=True)
        cnts = cnts + jnp.sum(m * valid, axis=1, keepdims=True)
        return (sums, cnts)

    sums, cnts = lax.fori_loop(
        0, _NROW2D, row,
        (jnp.zeros((G, 1), jnp.float32), jnp.zeros((G, 1), jnp.float32)))
    zsum = jnp.sum(zp_ref[...], axis=0)[:, None]
    cb3 = jnp.dot(b3_ref[...], wl_ref[...], preferred_element_type=jnp.float32)
    o_ref[...] = ((sums + zsum + cnts * cb3)
                  / jnp.maximum(cnts, 1.0)) + bl_ref[...]


def _final(batch2d, v2d, sn2d, zp, b3, wl, bl):
    return pl.pallas_call(
        _final_body,
        out_shape=jax.ShapeDtypeStruct((G, 1), jnp.float32),
    )(batch2d, v2d, sn2d, zp, b3, wl, bl)


def kernel(x, edge_index, edge_attr, batch, W1, b1, W2, b2, W3, b3, Wl, bl):
    src = edge_index[0].astype(jnp.int32)
    dst = edge_index[1].astype(jnp.int32)
    batch = batch.astype(jnp.int32)
    src3 = src.reshape(NW, NCHUNK, CW)
    dst3 = dst.reshape(NW, NCHUNK, CW)
    ew3 = edge_attr.reshape(NW, NCHUNK, CW)

    norm3, bidx3, selfn = _k0(src3, dst3, ew3, batch)
    sn1 = selfn.reshape(N_PAD, 1)
    norm1 = norm3.reshape(-1)
    bidx1 = bidx3.reshape(-1)

    npad = E_WP - E_W
    padidx = ((jnp.arange(npad)[None, :] + 317 * jnp.arange(NW)[:, None])
              % N_NODES).astype(jnp.int32)
    srcp = jnp.concatenate([src.reshape(NW, E_W), padidx], axis=1).reshape(-1)
    dstp = jnp.concatenate(
        [dst.reshape(NW, E_W), jnp.zeros((NW, npad), jnp.int32)],
        axis=1).reshape(-1)
    normp = jnp.concatenate(
        [norm1.reshape(NW, E_W), jnp.zeros((NW, npad), jnp.float32)],
        axis=1).reshape(-1)

    xp = jnp.pad(x, ((0, N_PAD - N_NODES), (0, 0)))
    hw1 = _mm(xp, W1)
    p = _agg(hw1, srcp, dstp, normp)
    hw2 = _comb_mm(p, hw1, sn1, b1.reshape(1, D), W2)
    q = _agg(hw2, srcp, dstp, normp)
    v = _head(q, hw2, sn1, b2.reshape(1, D), W3, Wl)

    zp = _pool(v.reshape(-1), src, norm1, bidx1)

    batch2d = jnp.pad(batch, (0, N_PAD - N_NODES)).reshape(_NROW2D, D)
    v2d = v.reshape(_NROW2D, D)
    sn2d = selfn.reshape(_NROW2D, D)
    return _final(batch2d, v2d, sn2d, zp, b3.reshape(1, D), Wl,
                  bl.reshape(1, 1))


# speedup vs baseline: 1.4058x; 1.4058x over previous
import functools

import jax
import jax.numpy as jnp
from jax import lax
from jax.experimental import pallas as pl
from jax.experimental.pallas import tpu as pltpu, tpu_sc as plsc

NC = 2
NS = 16
NW = NC * NS
LANES = 16

N_NODES = 10000
N_PAD = 10240
SEG = N_PAD // NS
E_TOT = 320000
E_W = E_TOT // NW
CW = 80
NCHUNK = E_W // CW
ROWS_W = N_PAD // NS
D = 128
G = 64
ROWBLK = 128

_mesh = plsc.VectorSubcoreMesh(core_axis_name="c", subcore_axis_name="s")


def _zero_vmem_1d(ref, n):
    z16 = jnp.zeros((LANES,), jnp.float32)

    def body(i, carry):
        ref[pl.ds(i * LANES, LANES)] = z16
        return carry

    lax.fori_loop(0, n // LANES, body, 0)


def _rsqrt16(x):
    i = plsc.bitcast(x, jnp.int32)
    y = plsc.bitcast(jnp.int32(0x5F3759DF) - (i >> 1), jnp.float32)
    for _ in range(4):
        y = y * (1.5 - 0.5 * x * y * y)
    return y


@functools.partial(
    pl.kernel,
    out_type=[
        jax.ShapeDtypeStruct((NW, NCHUNK, CW), jnp.float32),
        jax.ShapeDtypeStruct((NW, NCHUNK, CW), jnp.int32),
        jax.ShapeDtypeStruct((N_PAD,), jnp.float32),
    ],
    mesh=_mesh,
    compiler_params=pltpu.CompilerParams(needs_layout_passes=False),
    scratch_types=[
        pltpu.VMEM((NCHUNK, CW), jnp.int32),
        pltpu.VMEM((NCHUNK, CW), jnp.float32),
        pltpu.VMEM((N_PAD,), jnp.float32),
        pltpu.VMEM((SEG,), jnp.float32),
        pltpu.VMEM((SEG,), jnp.float32),
        pltpu.VMEM((N_PAD,), jnp.float32),
        pltpu.VMEM((N_NODES,), jnp.int32),
        pltpu.VMEM((NCHUNK, CW), jnp.int32),
        pltpu.VMEM((NCHUNK, CW), jnp.float32),
        pltpu.VMEM((NCHUNK, CW), jnp.int32),
        pltpu.VMEM_SHARED((NS, N_PAD), jnp.float32),
        pltpu.VMEM_SHARED((N_PAD,), jnp.float32),
    ],
)
def _k0(src3, dst3, ew3, batch_t, norm_o, bidx_o, selfn_o,
        dstA, ewA, degv, colacc, segbuf, dinvv, batchv,
        srcC, normv, bidxv, stage, dinv_sh):
    c = lax.axis_index("c")
    s = lax.axis_index("s")
    wid = c * NS + s

    _zero_vmem_1d(degv, N_PAD)

    def acc_deg(chunk_id):
        pltpu.sync_copy(dst3.at[chunk_id], dstA)
        pltpu.sync_copy(ew3.at[chunk_id], ewA)

        def body(j, carry):
            for k in range(CW // LANES):
                d16 = dstA[j, pl.ds(k * LANES, LANES)]
                w16 = ewA[j, pl.ds(k * LANES, LANES)]
                plsc.addupdate_scatter(degv, [d16], w16)
            return carry

        lax.fori_loop(0, NCHUNK, body, 0)

    acc_deg(s)
    acc_deg(s + NS)

    pltpu.sync_copy(degv, stage.at[s])
    plsc.subcore_barrier()
    _zero_vmem_1d(colacc, SEG)

    def reduce_tile(t, carry):
        pltpu.sync_copy(stage.at[t, pl.ds(s * SEG, SEG)], segbuf)
        for m in range(SEG // LANES):
            sl = pl.ds(m * LANES, LANES)
            colacc[sl] = colacc[sl] + segbuf[sl]
        return carry

    lax.fori_loop(0, NS, reduce_tile, 0)

    for m in range(SEG // LANES):
        sl = pl.ds(m * LANES, LANES)
        x = colacc[sl] + 1.0
        y = _rsqrt16(x)
        segbuf[sl] = y
        colacc[sl] = y * y

    pltpu.sync_copy(segbuf, dinv_sh.at[pl.ds(s * SEG, SEG)])

    @pl.when(c == 0)
    def _():
        pltpu.sync_copy(colacc, selfn_o.at[pl.ds(s * SEG, SEG)])

    plsc.subcore_barrier()
    pltpu.sync_copy(dinv_sh, dinvv)

    pltpu.sync_copy(batch_t, batchv)
    pltpu.sync_copy(src3.at[wid], srcC)
    pltpu.sync_copy(dst3.at[wid], dstA)
    pltpu.sync_copy(ew3.at[wid], ewA)

    def norm_body(j, carry):
        for k in range(CW // LANES):
            sl = pl.ds(k * LANES, LANES)
            s16 = srcC[j, sl]
            d16 = dstA[j, sl]
            w16 = ewA[j, sl]
            gs = plsc.load_gather(dinvv, [s16])
            gd = plsc.load_gather(dinvv, [d16])
            normv[j, sl] = gs * w16 * gd
            bidxv[j, sl] = plsc.load_gather(batchv, [d16])
        return carry

    lax.fori_loop(0, NCHUNK, norm_body, 0)
    pltpu.sync_copy(normv, norm_o.at[wid])
    pltpu.sync_copy(bidxv, bidx_o.at[wid])


CWA = 80
NCH_A = 126
E_WP = NCH_A * CWA
WB = 80


@functools.partial(
    pl.kernel,
    out_type=jax.ShapeDtypeStruct((NC, N_PAD, D), jnp.float32),
    mesh=_mesh,
    compiler_params=pltpu.CompilerParams(needs_layout_passes=False),
    scratch_types=(
        [pltpu.VMEM((NCH_A, CWA), jnp.int32)]
        + [pltpu.VMEM((CWA,), jnp.int32)] * 3
        + [pltpu.VMEM((CWA,), jnp.float32)] * 3
        + [pltpu.VMEM((CWA, D), jnp.float32)] * 3
        + [pltpu.SemaphoreType.DMA] * 9
        + [pltpu.VMEM_SHARED((N_PAD, D), jnp.float32)]
    ),
)
def _agg(hw, srcp, dstp3, normp, out,
         dstv, sw0, sw1, sw2, nw0, nw1, nw2, rw0, rw1, rw2,
         gs0, gs1, gs2, ss0, ss1, ss2, ws0, ws1, ws2, acc):
    c = lax.axis_index("c")
    s = lax.axis_index("s")
    wid = c * NS + s
    ebase = wid * E_WP
    bufs = [(sw0, nw0, rw0, gs0, ss0, ws0),
            (sw1, nw1, rw1, gs1, ss1, ws1),
            (sw2, nw2, rw2, gs2, ss2, ws2)]

    z16 = jnp.zeros((LANES,), jnp.float32)

    def zblk(j, carry):
        for k in range(D // LANES):
            rw0[j, pl.ds(k * LANES, LANES)] = z16
        return carry

    lax.fori_loop(0, CWA, zblk, 0)
    for i in range(ROWS_W // WB):
        pltpu.sync_copy(rw0, acc.at[pl.ds(s * ROWS_W + i * WB, WB)])
    pltpu.sync_copy(dstp3.at[wid], dstv)
    plsc.subcore_barrier()

    def win_async(j, sw, nw, ws):
        e0 = ebase + j * CWA
        pltpu.async_copy(srcp.at[pl.ds(e0, CWA)], sw, ws)
        pltpu.async_copy(normp.at[pl.ds(e0, CWA)], nw, ws)

    def win_wait(sw, nw, ws):
        pltpu.make_async_copy(srcp.at[pl.ds(0, CWA)], sw, ws).wait()
        pltpu.make_async_copy(normp.at[pl.ds(0, CWA)], nw, ws).wait()

    def scale(rows, nw):
        @plsc.parallel_loop(0, CWA // LANES)
        def grp(g):
            nm16 = nw[pl.ds(g * LANES, LANES)]
            for l in range(LANES):
                e = g * LANES + l
                sc = jnp.broadcast_to(nm16[l], (LANES,))
                for k in range(D // LANES):
                    sl = pl.ds(k * LANES, LANES)
                    rows[e, sl] = rows[e, sl] * sc

    for t in range(3):
        sw, nw, rw, gs, ss, ws = bufs[t]
        win_async(t, sw, nw, ws)
    for t in range(2):
        sw, nw, rw, gs, ss, ws = bufs[t]
        win_wait(sw, nw, ws)
        pltpu.async_copy(hw.at[sw], rw, gs)

    def body(i, carry):
        for t in range(3):
            sw, nw, rw, gs, ss, ws = bufs[t]
            j = 3 * i + t
            pltpu.make_async_copy(hw.at[sw], rw, gs).wait()
            scale(rw, nw)
            pltpu.async_copy(rw, acc.at[dstv.at[j]], ss, add=True)

            @pl.when(j + 3 < NCH_A)
            def _():
                win_async(j + 3, sw, nw, ws)

            nt = (t + 2) % 3
            swn, nwn, rwn, gsn, ssn, wsn = bufs[nt]
            if t == 0:
                @pl.when(i > 0)
                def _():
                    pltpu.make_async_copy(rwn, acc.at[dstv.at[0]], ssn).wait()

                win_wait(swn, nwn, wsn)
                pltpu.async_copy(hw.at[swn], rwn, gsn)
            else:
                @pl.when(j + 2 < NCH_A)
                def _():
                    pltpu.make_async_copy(rwn, acc.at[dstv.at[0]], ssn).wait()
                    win_wait(swn, nwn, wsn)
                    pltpu.async_copy(hw.at[swn], rwn, gsn)
        return carry

    lax.fori_loop(0, NCH_A // 3, body, 0)
    for t in range(3):
        sw, nw, rw, gs, ss, ws = bufs[t]
        pltpu.make_async_copy(rw, acc.at[dstv.at[0]], ss).wait()
    plsc.subcore_barrier()

    for i in range(ROWS_W // WB):
        r0 = s * ROWS_W + i * WB
        pltpu.sync_copy(acc.at[pl.ds(r0, WB)], rw0)
        pltpu.sync_copy(rw0, out.at[c, pl.ds(r0, WB)])


@functools.partial(
    pl.kernel,
    out_type=jax.ShapeDtypeStruct((NW, G), jnp.float32),
    mesh=_mesh,
    compiler_params=pltpu.CompilerParams(needs_layout_passes=False),
    scratch_types=[
        pltpu.VMEM((N_PAD,), jnp.float32),
        pltpu.VMEM((E_W,), jnp.int32),
        pltpu.VMEM((E_W,), jnp.float32),
        pltpu.VMEM((E_W,), jnp.int32),
        pltpu.VMEM((G,), jnp.float32),
    ],
)
def _pool(vtab, src1, norm1, bidx1, out, vv, srcv, normv, bidxv, bins):
    c = lax.axis_index("c")
    s = lax.axis_index("s")
    wid = c * NS + s
    ebase = wid * E_W
    pltpu.sync_copy(vtab, vv)
    pltpu.sync_copy(src1.at[pl.ds(ebase, E_W)], srcv)
    pltpu.sync_copy(norm1.at[pl.ds(ebase, E_W)], normv)
    pltpu.sync_copy(bidx1.at[pl.ds(ebase, E_W)], bidxv)
    _zero_vmem_1d(bins, G)

    def body(j, carry):
        sl = pl.ds(j * LANES, LANES)
        s16 = srcv[sl]
        nm = normv[sl]
        b16 = bidxv[sl]
        gv = plsc.load_gather(vv, [s16])
        plsc.addupdate_scatter(bins, [b16], gv * nm)
        return carry

    lax.fori_loop(0, E_W // LANES, body, 0)
    pltpu.sync_copy(bins, out.at[wid])


_RB = 2048
_NG = N_PAD // _RB


def _mm_body(x_ref, w_ref, o_ref):
    o_ref[...] = jnp.dot(x_ref[...], w_ref[...],
                         preferred_element_type=jnp.float32)


def _mm(x, w):
    return pl.pallas_call(
        _mm_body,
        grid=(_NG,),
        in_specs=[
            pl.BlockSpec((_RB, D), lambda i: (i, 0)),
            pl.BlockSpec((D, D), lambda i: (0, 0)),
        ],
        out_specs=pl.BlockSpec((_RB, D), lambda i: (i, 0)),
        out_shape=jax.ShapeDtypeStruct((N_PAD, D), jnp.float32),
    )(x, w)


def _comb_body(p_ref, hw_ref, sn_ref, b_ref, w_ref, o_ref):
    h = p_ref[0] + p_ref[1] + hw_ref[...] * sn_ref[...] + b_ref[...]
    h = jnp.maximum(h, 0.0)
    o_ref[...] = jnp.dot(h, w_ref[...], preferred_element_type=jnp.float32)


def _comb_mm(p, hw, sn, b, w):
    return pl.pallas_call(
        _comb_body,
        grid=(_NG,),
        in_specs=[
            pl.BlockSpec((NC, _RB, D), lambda i: (0, i, 0)),
            pl.BlockSpec((_RB, D), lambda i: (i, 0)),
            pl.BlockSpec((_RB, 1), lambda i: (i, 0)),
            pl.BlockSpec((1, D), lambda i: (0, 0)),
            pl.BlockSpec((D, D), lambda i: (0, 0)),
        ],
        out_specs=pl.BlockSpec((_RB, D), lambda i: (i, 0)),
        out_shape=jax.ShapeDtypeStruct((N_PAD, D), jnp.float32),
    )(p, hw, sn, b, w)


def _head_body(q_ref, hw_ref, sn_ref, b_ref, w3_ref, wl_ref, o_ref):
    h = q_ref[0] + q_ref[1] + hw_ref[...] * sn_ref[...] + b_ref[...]
    h = jnp.maximum(h, 0.0)
    w3l = jnp.dot(w3_ref[...], wl_ref[...], preferred_element_type=jnp.float32)
    o_ref[...] = jnp.dot(h, w3l, preferred_element_type=jnp.float32)


def _head(q, hw, sn, b, w3, wl):
    return pl.pallas_call(
        _head_body,
        grid=(_NG,),
        in_specs=[
            pl.BlockSpec((NC, _RB, D), lambda i: (0, i, 0)),
            pl.BlockSpec((_RB, D), lambda i: (i, 0)),
            pl.BlockSpec((_RB, 1), lambda i: (i, 0)),
            pl.BlockSpec((1, D), lambda i: (0, 0)),
            pl.BlockSpec((D, D), lambda i: (0, 0)),
            pl.BlockSpec((D, 1), lambda i: (0, 0)),
        ],
        out_specs=pl.BlockSpec((_RB, 1), lambda i: (i, 0)),
        out_shape=jax.ShapeDtypeStruct((N_PAD, 1), jnp.float32),
    )(q, hw, sn, b, w3, wl)


_NROW2D = N_PAD // D


def _final_body(batch_ref, v_ref, sn_ref, zp_ref, b3_ref, wl_ref, bl_ref,
                o_ref):
    giota = lax.broadcasted_iota(jnp.int32, (G, 1), 0)
    liota = lax.broadcasted_iota(jnp.int32, (1, D), 1)

    def row(r, carry):
        sums, cnts = carry
        brow = batch_ref[pl.ds(r, 1), :]
        wrow = v_ref[pl.ds(r, 1), :] * sn_ref[pl.ds(r, 1), :]
        valid = ((liota + r * D) < N_NODES).astype(jnp.float32)
        m = (brow == giota).astype(jnp.float32)
        sums = sums + jnp.sum(m * wrow, axis=1, keepdims=True)
        cnts = cnts + jnp.sum(m * valid, axis=1, keepdims=True)
        return (sums, cnts)

    sums, cnts = lax.fori_loop(
        0, _NROW2D, row,
        (jnp.zeros((G, 1), jnp.float32), jnp.zeros((G, 1), jnp.float32)))
    zsum = jnp.sum(zp_ref[...], axis=0)[:, None]
    cb3 = jnp.dot(b3_ref[...], wl_ref[...], preferred_element_type=jnp.float32)
    o_ref[...] = ((sums + zsum + cnts * cb3)
                  / jnp.maximum(cnts, 1.0)) + bl_ref[...]


def _final(batch2d, v2d, sn2d, zp, b3, wl, bl):
    return pl.pallas_call(
        _final_body,
        out_shape=jax.ShapeDtypeStruct((G, 1), jnp.float32),
    )(batch2d, v2d, sn2d, zp, b3, wl, bl)


def kernel(x, edge_index, edge_attr, batch, W1, b1, W2, b2, W3, b3, Wl, bl):
    src = edge_index[0].astype(jnp.int32)
    dst = edge_index[1].astype(jnp.int32)
    batch = batch.astype(jnp.int32)
    src3 = src.reshape(NW, NCHUNK, CW)
    dst3 = dst.reshape(NW, NCHUNK, CW)
    ew3 = edge_attr.reshape(NW, NCHUNK, CW)

    norm3, bidx3, selfn = _k0(src3, dst3, ew3, batch)
    sn1 = selfn.reshape(N_PAD, 1)
    norm1 = norm3.reshape(-1)
    bidx1 = bidx3.reshape(-1)

    npad = E_WP - E_W
    padidx = ((jnp.arange(npad)[None, :] + 317 * jnp.arange(NW)[:, None])
              % N_NODES).astype(jnp.int32)
    srcp = jnp.concatenate([src.reshape(NW, E_W), padidx], axis=1).reshape(-1)
    dstp3 = jnp.concatenate(
        [dst.reshape(NW, E_W), jnp.zeros((NW, npad), jnp.int32)],
        axis=1).reshape(NW, NCH_A, CWA)
    normp = jnp.concatenate(
        [norm1.reshape(NW, E_W), jnp.zeros((NW, npad), jnp.float32)],
        axis=1).reshape(-1)

    xp = jnp.pad(x, ((0, N_PAD - N_NODES), (0, 0)))
    hw1 = _mm(xp, W1)
    p = _agg(hw1, srcp, dstp3, normp)
    hw2 = _comb_mm(p, hw1, sn1, b1.reshape(1, D), W2)
    q = _agg(hw2, srcp, dstp3, normp)
    v = _head(q, hw2, sn1, b2.reshape(1, D), W3, Wl)

    zp = _pool(v.reshape(-1), src, norm1, bidx1)

    batch2d = jnp.pad(batch, (0, N_PAD - N_NODES)).reshape(_NROW2D, D)
    v2d = v.reshape(_NROW2D, D)
    sn2d = selfn.reshape(_NROW2D, D)
    return _final(batch2d, v2d, sn2d, zp, b3.reshape(1, D), Wl,
                  bl.reshape(1, 1))
